# baseline (speedup 1.0000x reference)
import jax
import jax.numpy as jnp
from jax import lax
from jax.experimental import pallas as pl
from jax.experimental.pallas import tpu as pltpu, tpu_sc as plsc

N_NEURONS = 10000
N_BKG = 100
NNZ = 160000
N_SYN = 5
BKG_RATE = 250

NC, NS = 2, 16
NW = NC * NS
K = 160
NPL = K // 16
N_PAD = 10240
NCHUNK = N_PAD // K
CHUNKS_PER_W = NCHUNK // NW
BLKN = 3072
NBIN = 1040
WROW = K * N_SYN
WCOL = 128
PPT = NNZ // NS
PPW = PPT // 2
ASTEP = (PPW + 15) // 16


def _bin10(r):
    return lax.shift_right_logical(r * 52429, 19)


def _sc_body(idx_hbm, w_hbm, tau_hbm, zeros_hbm, out_hbm, shared):
    cid = lax.axis_index("c")
    sid = lax.axis_index("s")
    wid = sid * NC + cid
    lane = lax.iota(jnp.int32, 16)

    if True:
        def phase_a(idx2_v, rp_loc):
            fullv = jnp.full((16,), NNZ, jnp.int32)
            def init(v, _):
                rp_loc[pl.ds(v * 16, 16)] = fullv
                return 0
            lax.fori_loop(0, NBIN // 16, init, 0)

            for h in range(2):
                pbase = sid * PPT + h * PPW
                pend = pbase + PPW
                g0p = pl.multiple_of(jnp.maximum(pbase - 8, 0), 8)
                pltpu.sync_copy(
                    idx_hbm.at[pl.ds(pl.multiple_of(g0p * 2, 8),
                                     2 * (PPW + 8))], idx2_v)

                def scan(i, _, g0p=g0p, pbase=pbase, pend=pend):
                    p = pbase + i * 16 + lane
                    lpc = jnp.minimum(p - g0p, PPW + 7)
                    r = plsc.load_gather(idx2_v, [2 * lpc])
                    rprev = plsc.load_gather(
                        idx2_v, [jnp.maximum(2 * lpc - 2, 0)])
                    q = _bin10(r)
                    qprev = jnp.where(p == 0, -1, _bin10(rprev))
                    m = (q != qprev) & (p < pend)
                    plsc.store_scatter(rp_loc, [q], p, mask=m)
                    return 0

                lax.fori_loop(0, ASTEP, scan, 0)

            pltpu.sync_copy(
                rp_loc,
                shared.at[pl.ds(pl.multiple_of(sid * NBIN, 8), NBIN)])

        pl.run_scoped(phase_a,
                      pltpu.VMEM((2 * (PPW + 8),), jnp.int32),
                      pltpu.VMEM((NBIN,), jnp.int32))
        plsc.subcore_barrier()

        @pl.when(sid == 0)
        def _reduce():
            def phase_b(stage_v, fin_v):
                pltpu.sync_copy(shared, stage_v)

                def redv(v, _):
                    acc = stage_v[pl.ds(v * 16, 16)]
                    for t in range(1, NS):
                        acc = jnp.minimum(
                            acc, stage_v[pl.ds(t * NBIN + v * 16, 16)])
                    fin_v[pl.ds(v * 16, 16)] = acc
                    return 0
                lax.fori_loop(0, NBIN // 16, redv, 0)

                def sufv(i, carry):
                    v = NBIN // 16 - 1 - i
                    seg = fin_v[pl.ds(v * 16, 16)]
                    rs = lax.rev(seg, (0,))
                    run = -plsc.cummax(-rs)
                    comb = jnp.minimum(run, carry)
                    fin_v[pl.ds(v * 16, 16)] = lax.rev(comb, (0,))
                    return jnp.min(comb)
                lax.fori_loop(0, NBIN // 16, sufv, jnp.int32(NNZ))

                pltpu.sync_copy(fin_v, shared.at[pl.ds(0, NBIN)])

            pl.run_scoped(phase_b,
                          pltpu.VMEM((NS * NBIN,), jnp.int32),
                          pltpu.VMEM((NBIN,), jnp.int32))

        plsc.subcore_barrier()

        def phase_c(idx_v, w_v, tau_v, rp_v, wl_v):
            for it in range(CHUNKS_PER_W):
                chunk = wid + NW * it
                n0 = chunk * K

                pltpu.sync_copy(
                    shared.at[pl.ds(pl.multiple_of(chunk * 16, 8), 24)],
                    rp_v)
                a = plsc.load_gather(rp_v, [lane])
                b = plsc.load_gather(rp_v, [lane + 1])
                p1 = jnp.max(b)

                pltpu.sync_copy(zeros_hbm, wl_v)

                ws0 = jnp.min(a) & ~jnp.int32(7)

                def window(ws_carry, a=a, b=b, n0=n0):
                    ws = pl.multiple_of(ws_carry, 8)
                    we = ws + BLKN
                    wd = pl.multiple_of(jnp.minimum(ws, NNZ - BLKN), 8)
                    pltpu.sync_copy(
                        idx_hbm.at[pl.ds(pl.multiple_of(wd * 2, 8),
                                         2 * BLKN)], idx_v)
                    pltpu.sync_copy(w_hbm.at[pl.ds(wd, BLKN)], w_v)
                    pltpu.sync_copy(
                        tau_hbm.at[pl.ds(pl.multiple_of(wd * 5, 8),
                                         5 * BLKN)], tau_v)
                    c0 = jnp.maximum(a, ws)
                    bmin = jnp.minimum(b, we)
                    steps = jnp.max(jnp.maximum(bmin - c0, 0))

                    def step(i, _, c0=c0, bmin=bmin, wd=wd, n0=n0):
                        ci = c0 + i
                        m = ci < bmin
                        off = jnp.minimum(ci - wd, BLKN - 1)
                        r16 = plsc.load_gather(idx_v, [off * 2])
                        c16 = plsc.load_gather(idx_v, [off * 2 + 1])
                        w16 = plsc.load_gather(w_v, [off])
                        row = (r16 - n0) * N_SYN
                        toff = off * N_SYN
                        for s in range(N_SYN):
                            t16 = plsc.load_gather(tau_v, [toff + s])
                            plsc.addupdate_scatter(
                                wl_v, [row + s, c16], w16 * t16, mask=m)
                        return 0

                    lax.fori_loop(0, steps, step, 0)
                    return ws + BLKN

                lax.while_loop(lambda ws, p1=p1: ws < p1, window, ws0)

                pltpu.sync_copy(
                    wl_v,
                    out_hbm.at[pl.ds(pl.multiple_of(chunk * WROW, 8), WROW)])

        pl.run_scoped(phase_c,
                      pltpu.VMEM((2 * BLKN,), jnp.int32),
                      pltpu.VMEM((BLKN,), jnp.float32),
                      pltpu.VMEM((N_SYN * BLKN,), jnp.float32),
                      pltpu.VMEM((24,), jnp.int32),
                      pltpu.VMEM((WROW, WCOL), jnp.float32))

_sc_scatter = pl.kernel(
    _sc_body,
    out_type=jax.ShapeDtypeStruct((NCHUNK * WROW, WCOL), jnp.float32),
    mesh=plsc.VectorSubcoreMesh(core_axis_name="c", subcore_axis_name="s",
                                num_cores=NC, num_subcores=NS),
    compiler_params=pltpu.CompilerParams(needs_layout_passes=False),
    scratch_types=[pltpu.VMEM_SHARED((NS * NBIN,), jnp.int32)],
)


def _mm_body(s_ref, w_ref, o_ref):
    o_ref[0] = lax.dot_general(
        s_ref[...], w_ref[...], (((1,), (1,)), ((), ())),
        preferred_element_type=jnp.float32)


_BN = 2048


@jax.jit
def kernel(inp, indices, weights, tau_syn_weights):
    bsz, t, _ = inp.shape
    bt = bsz * t

    spikes = (jax.random.uniform(jax.random.key(42), (bsz, t, N_BKG))
              < BKG_RATE * 0.001).astype(jnp.float32).reshape(bt, N_BKG)
    spikes = jnp.pad(spikes, ((0, 0), (0, WCOL - N_BKG)))

    idx_flat = indices.astype(jnp.int32).reshape(2 * NNZ)
    w32 = weights.astype(jnp.float32)
    tau_flat = tau_syn_weights.astype(jnp.float32).reshape(N_SYN * NNZ)
    zeros = jnp.zeros((WROW, WCOL), jnp.float32)

    w2 = _sc_scatter(idx_flat, w32, tau_flat, zeros)

    nout = N_NEURONS * N_SYN
    out = pl.pallas_call(
        _mm_body,
        grid=((nout + _BN - 1) // _BN,),
        in_specs=[
            pl.BlockSpec((bt, WCOL), lambda j: (0, 0)),
            pl.BlockSpec((_BN, WCOL), lambda j: (j, 0)),
        ],
        out_specs=pl.BlockSpec((1, bt, _BN), lambda j: (0, 0, j)),
        out_shape=jax.ShapeDtypeStruct((bsz, bt, nout), jnp.float32),
    )(spikes, w2)

    return out


# speedup vs baseline: 1.1597x; 1.1597x over previous
import jax
import jax.numpy as jnp
from jax import lax
from jax.experimental import pallas as pl
from jax.experimental.pallas import tpu as pltpu, tpu_sc as plsc

N_NEURONS = 10000
N_BKG = 100
NNZ = 160000
N_SYN = 5
BKG_RATE = 250

NC, NS = 2, 16
NW = NC * NS
K = 160
NPL = K // 16
N_PAD = 10240
NCHUNK = N_PAD // K
CHUNKS_PER_W = NCHUNK // NW
BLKN = 3072
NBIN = 1040
WROW = K * N_SYN
WCOL = 128
PPT = NNZ // NS
PPW = PPT // 2
ASTEP = (PPW + 15) // 16


def _bin10(r):
    return lax.shift_right_logical(r * 52429, 19)


def _sc_body(idx_hbm, w_hbm, tau_hbm, zeros_hbm, out_hbm, shared):
    cid = lax.axis_index("c")
    sid = lax.axis_index("s")
    wid = sid * NC + cid
    lane = lax.iota(jnp.int32, 16)

    if True:
        def phase_a(idx2_v, rp_loc):
            fullv = jnp.full((16,), NNZ, jnp.int32)
            def init(v, _):
                rp_loc[pl.ds(v * 16, 16)] = fullv
                return 0
            lax.fori_loop(0, NBIN // 16, init, 0)

            for h in range(2):
                pbase = sid * PPT + h * PPW
                pend = pbase + PPW
                g0p = pl.multiple_of(jnp.maximum(pbase - 8, 0), 8)
                pltpu.sync_copy(
                    idx_hbm.at[pl.ds(pl.multiple_of(g0p * 2, 8),
                                     2 * (PPW + 8))], idx2_v)

                def scan(i, _, g0p=g0p, pbase=pbase, pend=pend):
                    p = pbase + i * 16 + lane
                    lpc = jnp.minimum(p - g0p, PPW + 7)
                    r = plsc.load_gather(idx2_v, [2 * lpc])
                    rprev = plsc.load_gather(
                        idx2_v, [jnp.maximum(2 * lpc - 2, 0)])
                    q = _bin10(r)
                    qprev = jnp.where(p == 0, -1, _bin10(rprev))
                    m = (q != qprev) & (p < pend)
                    plsc.store_scatter(rp_loc, [q], p, mask=m)
                    return 0

                lax.fori_loop(0, ASTEP, scan, 0)

            pltpu.sync_copy(
                rp_loc,
                shared.at[pl.ds(pl.multiple_of(sid * NBIN, 8), NBIN)])

        pl.run_scoped(phase_a,
                      pltpu.VMEM((2 * (PPW + 8),), jnp.int32),
                      pltpu.VMEM((NBIN,), jnp.int32))
        plsc.subcore_barrier()

        @pl.when(sid == 0)
        def _reduce():
            def phase_b(stage_v, fin_v):
                pltpu.sync_copy(shared, stage_v)

                def redv(v, _):
                    acc = stage_v[pl.ds(v * 16, 16)]
                    for t in range(1, NS):
                        acc = jnp.minimum(
                            acc, stage_v[pl.ds(t * NBIN + v * 16, 16)])
                    fin_v[pl.ds(v * 16, 16)] = acc
                    return 0
                lax.fori_loop(0, NBIN // 16, redv, 0)

                def sufv(i, carry):
                    v = NBIN // 16 - 1 - i
                    seg = fin_v[pl.ds(v * 16, 16)]
                    rs = lax.rev(seg, (0,))
                    run = -plsc.cummax(-rs)
                    comb = jnp.minimum(run, carry)
                    fin_v[pl.ds(v * 16, 16)] = lax.rev(comb, (0,))
                    return jnp.min(comb)
                lax.fori_loop(0, NBIN // 16, sufv, jnp.int32(NNZ))

                pltpu.sync_copy(fin_v, shared.at[pl.ds(0, NBIN)])

            pl.run_scoped(phase_b,
                          pltpu.VMEM((NS * NBIN,), jnp.int32),
                          pltpu.VMEM((NBIN,), jnp.int32))

        plsc.subcore_barrier()

        def phase_c(idx_v, w_v, tau_v, rp_v, wl_v):
            for it in range(CHUNKS_PER_W):
                chunk = wid + NW * it
                n0 = chunk * K

                pltpu.sync_copy(
                    shared.at[pl.ds(pl.multiple_of(chunk * 16, 8), 24)],
                    rp_v)
                a = plsc.load_gather(rp_v, [lane])
                b = plsc.load_gather(rp_v, [lane + 1])
                p1 = jnp.max(b)

                pltpu.sync_copy(zeros_hbm, wl_v)

                ws0 = jnp.min(a) & ~jnp.int32(7)

                def window(ws_carry, a=a, b=b, n0=n0):
                    ws = pl.multiple_of(ws_carry, 8)
                    we = ws + BLKN
                    wd = pl.multiple_of(jnp.minimum(ws, NNZ - BLKN), 8)
                    pltpu.sync_copy(
                        idx_hbm.at[pl.ds(pl.multiple_of(wd * 2, 8),
                                         2 * BLKN)], idx_v)
                    pltpu.sync_copy(w_hbm.at[pl.ds(wd, BLKN)], w_v)
                    pltpu.sync_copy(
                        tau_hbm.at[pl.ds(pl.multiple_of(wd * 5, 8),
                                         5 * BLKN)], tau_v)
                    c0 = jnp.maximum(a, ws)
                    bmin = jnp.minimum(b, we)
                    steps = jnp.max(jnp.maximum(bmin - c0, 0))

                    def step(i, _, c0=c0, bmin=bmin, wd=wd, n0=n0):
                        ci = c0 + i
                        m = ci < bmin
                        off = jnp.minimum(ci - wd, BLKN - 1)
                        r16 = plsc.load_gather(idx_v, [off * 2])
                        c16 = plsc.load_gather(idx_v, [off * 2 + 1])
                        w16 = plsc.load_gather(w_v, [off])
                        row = (r16 - n0) * N_SYN
                        toff = off * N_SYN
                        for s in range(N_SYN):
                            t16 = plsc.load_gather(tau_v, [toff + s])
                            plsc.addupdate_scatter(
                                wl_v, [row + s, c16], w16 * t16, mask=m)
                        return 0

                    lax.fori_loop(0, steps, step, 0)
                    return ws + BLKN

                lax.while_loop(lambda ws, p1=p1: ws < p1, window, ws0)

                pltpu.sync_copy(
                    wl_v,
                    out_hbm.at[pl.ds(pl.multiple_of(chunk * WROW, 8), WROW)])

        pl.run_scoped(phase_c,
                      pltpu.VMEM((2 * BLKN,), jnp.int32),
                      pltpu.VMEM((BLKN,), jnp.float32),
                      pltpu.VMEM((N_SYN * BLKN,), jnp.float32),
                      pltpu.VMEM((24,), jnp.int32),
                      pltpu.VMEM((WROW, WCOL), jnp.float32))

_sc_scatter = pl.kernel(
    _sc_body,
    out_type=jax.ShapeDtypeStruct((NCHUNK * WROW, WCOL), jnp.float32),
    mesh=plsc.VectorSubcoreMesh(core_axis_name="c", subcore_axis_name="s",
                                num_cores=NC, num_subcores=NS),
    compiler_params=pltpu.CompilerParams(needs_layout_passes=False),
    scratch_types=[pltpu.VMEM_SHARED((NS * NBIN,), jnp.int32)],
)


def _mm_body(s_ref, w_ref, o_ref):
    o_ref[...] = lax.dot_general(
        w_ref[...], s_ref[...], (((1,), (1,)), ((), ())),
        preferred_element_type=jnp.float32)


_BN = 2048


@jax.jit
def kernel(inp, indices, weights, tau_syn_weights):
    bsz, t, _ = inp.shape
    bt = bsz * t

    spikes = (jax.random.uniform(jax.random.key(42), (bsz, t, N_BKG))
              < BKG_RATE * 0.001).astype(jnp.float32).reshape(bt, N_BKG)
    spikes = jnp.pad(spikes, ((0, 0), (0, WCOL - N_BKG)))

    idx_flat = indices.astype(jnp.int32).reshape(2 * NNZ)
    w32 = weights.astype(jnp.float32)
    tau_flat = tau_syn_weights.astype(jnp.float32).reshape(N_SYN * NNZ)
    zeros = jnp.zeros((WROW, WCOL), jnp.float32)

    w2 = _sc_scatter(idx_flat, w32, tau_flat, zeros)

    nout = N_NEURONS * N_SYN
    out = pl.pallas_call(
        _mm_body,
        grid=((nout + _BN - 1) // _BN,),
        in_specs=[
            pl.BlockSpec((bt, WCOL), lambda j: (0, 0)),
            pl.BlockSpec((_BN, WCOL), lambda j: (j, 0)),
        ],
        out_specs=pl.BlockSpec((_BN, bt), lambda j: (j, 0)),
        out_shape=jax.ShapeDtypeStruct((nout, bt), jnp.float32),
    )(spikes, w2)

    return jnp.transpose(out).reshape(bsz, bt, nout)


# speedup vs baseline: 1.6200x; 1.3970x over previous
import jax
import jax.numpy as jnp
from jax import lax
from jax.experimental import pallas as pl
from jax.experimental.pallas import tpu as pltpu, tpu_sc as plsc

N_NEURONS = 10000
N_BKG = 100
NNZ = 160000
N_SYN = 5
BKG_RATE = 250

NC, NS = 2, 16
NW = NC * NS
K = 160
NPL = K // 16
N_PAD = 10240
NCHUNK = N_PAD // K
CHUNKS_PER_W = NCHUNK // NW
BLKN = 2432
NBIN = 1040
WROW = K * N_SYN
WCOL = 128
PPT = NNZ // NS
PPW = PPT // 2
ASTEP = (PPW + 15) // 16


def _bin10(r):
    return lax.shift_right_logical(r * 52429, 19)


def _sc_body(idx_hbm, w_hbm, tau_hbm, zeros_hbm, out_hbm, shared):
    cid = lax.axis_index("c")
    sid = lax.axis_index("s")
    wid = sid * NC + cid
    lane = lax.iota(jnp.int32, 16)

    if True:
        def phase_a(idx2_v, rp_loc):
            fullv = jnp.full((16,), NNZ, jnp.int32)
            def init(v, _):
                rp_loc[pl.ds(v * 16, 16)] = fullv
                return 0
            lax.fori_loop(0, NBIN // 16, init, 0)

            for h in range(2):
                pbase = sid * PPT + h * PPW
                pend = pbase + PPW
                g0p = pl.multiple_of(jnp.maximum(pbase - 8, 0), 8)
                pltpu.sync_copy(
                    idx_hbm.at[pl.ds(pl.multiple_of(g0p * 2, 8),
                                     2 * (PPW + 8))], idx2_v)

                def scan(i, _, g0p=g0p, pbase=pbase, pend=pend):
                    p = pbase + i * 16 + lane
                    lpc = jnp.minimum(p - g0p, PPW + 7)
                    r = plsc.load_gather(idx2_v, [2 * lpc])
                    rprev = plsc.load_gather(
                        idx2_v, [jnp.maximum(2 * lpc - 2, 0)])
                    q = _bin10(r)
                    qprev = jnp.where(p == 0, -1, _bin10(rprev))
                    m = (q != qprev) & (p < pend)
                    plsc.store_scatter(rp_loc, [q], p, mask=m)
                    return 0

                lax.fori_loop(0, ASTEP, scan, 0)

            pltpu.sync_copy(
                rp_loc,
                shared.at[pl.ds(pl.multiple_of(sid * NBIN, 8), NBIN)])

        pl.run_scoped(phase_a,
                      pltpu.VMEM((2 * (PPW + 8),), jnp.int32),
                      pltpu.VMEM((NBIN,), jnp.int32))
        plsc.subcore_barrier()

        @pl.when(sid == 0)
        def _reduce():
            def phase_b(stage_v, fin_v):
                pltpu.sync_copy(shared, stage_v)

                def redv(v, _):
                    acc = stage_v[pl.ds(v * 16, 16)]
                    for t in range(1, NS):
                        acc = jnp.minimum(
                            acc, stage_v[pl.ds(t * NBIN + v * 16, 16)])
                    fin_v[pl.ds(v * 16, 16)] = acc
                    return 0
                lax.fori_loop(0, NBIN // 16, redv, 0)

                def sufv(i, carry):
                    v = NBIN // 16 - 1 - i
                    seg = fin_v[pl.ds(v * 16, 16)]
                    rs = lax.rev(seg, (0,))
                    run = -plsc.cummax(-rs)
                    comb = jnp.minimum(run, carry)
                    fin_v[pl.ds(v * 16, 16)] = lax.rev(comb, (0,))
                    return jnp.min(comb)
                lax.fori_loop(0, NBIN // 16, sufv, jnp.int32(NNZ))

                pltpu.sync_copy(fin_v, shared.at[pl.ds(0, NBIN)])

            pl.run_scoped(phase_b,
                          pltpu.VMEM((NS * NBIN,), jnp.int32),
                          pltpu.VMEM((NBIN,), jnp.int32))

        plsc.subcore_barrier()

        def phase_c(idx_v, w_v, tau_v, rp_v, wl_v):
            for it in range(CHUNKS_PER_W):
                chunk = wid + NW * it
                n0 = chunk * K

                pltpu.sync_copy(
                    shared.at[pl.ds(pl.multiple_of(chunk * 16, 8), 24)],
                    rp_v)
                a = plsc.load_gather(rp_v, [lane])
                b = plsc.load_gather(rp_v, [lane + 1])
                p1 = jnp.max(b)

                pltpu.sync_copy(zeros_hbm, wl_v)

                ws0 = jnp.min(a) & ~jnp.int32(127)

                def window(ws_carry, a=a, b=b, n0=n0):
                    ws = pl.multiple_of(ws_carry, 128)
                    we = ws + BLKN
                    wd = pl.multiple_of(jnp.minimum(ws, NNZ - BLKN), 128)
                    pltpu.sync_copy(
                        idx_hbm.at[pl.ds(pl.multiple_of(wd * 2, 8),
                                         2 * BLKN)], idx_v)
                    pltpu.sync_copy(w_hbm.at[pl.ds(wd, BLKN)], w_v)
                    pltpu.sync_copy(tau_hbm.at[:, pl.ds(wd, BLKN)], tau_v)
                    c0 = jnp.maximum(a, ws)
                    bmin = jnp.minimum(b, we)
                    steps = jnp.max(jnp.maximum(bmin - c0, 0))

                    def step(i, _, c0=c0, bmin=bmin, wd=wd, n0=n0):
                        ci = c0 + i
                        m = ci < bmin
                        off = jnp.minimum(ci - wd, BLKN - 1)
                        r16 = plsc.load_gather(idx_v, [off * 2])
                        c16 = plsc.load_gather(idx_v, [off * 2 + 1])
                        w16 = plsc.load_gather(w_v, [off])
                        row = (r16 - n0) * N_SYN
                        for s in range(N_SYN):
                            t16 = plsc.load_gather(
                                tau_v, [jnp.full((16,), s, jnp.int32), off])
                            plsc.addupdate_scatter(
                                wl_v, [row + s, c16], w16 * t16, mask=m)
                        return 0

                    lax.fori_loop(0, steps, step, 0)
                    return ws + BLKN

                lax.while_loop(lambda ws, p1=p1: ws < p1, window, ws0)

                pltpu.sync_copy(
                    wl_v,
                    out_hbm.at[pl.ds(pl.multiple_of(chunk * WROW, 8), WROW)])

        pl.run_scoped(phase_c,
                      pltpu.VMEM((2 * BLKN,), jnp.int32),
                      pltpu.VMEM((BLKN,), jnp.float32),
                      pltpu.VMEM((N_SYN, BLKN), jnp.float32),
                      pltpu.VMEM((24,), jnp.int32),
                      pltpu.VMEM((WROW, WCOL), jnp.float32))

_sc_scatter = pl.kernel(
    _sc_body,
    out_type=jax.ShapeDtypeStruct((NCHUNK * WROW, WCOL), jnp.float32),
    mesh=plsc.VectorSubcoreMesh(core_axis_name="c", subcore_axis_name="s",
                                num_cores=NC, num_subcores=NS),
    compiler_params=pltpu.CompilerParams(needs_layout_passes=False),
    scratch_types=[pltpu.VMEM_SHARED((NS * NBIN,), jnp.int32)],
)


def _mm_body(s_ref, w_ref, o_ref):
    o_ref[...] = lax.dot_general(
        w_ref[...], s_ref[...], (((1,), (1,)), ((), ())),
        preferred_element_type=jnp.float32)


_BN = 2048


@jax.jit
def kernel(inp, indices, weights, tau_syn_weights):
    bsz, t, _ = inp.shape
    bt = bsz * t

    spikes = (jax.random.uniform(jax.random.key(42), (bsz, t, N_BKG))
              < BKG_RATE * 0.001).astype(jnp.float32).reshape(bt, N_BKG)
    spikes = jnp.pad(spikes, ((0, 0), (0, WCOL - N_BKG)))

    idx_flat = indices.astype(jnp.int32).reshape(2 * NNZ)
    w32 = weights.astype(jnp.float32)
    tau_t = tau_syn_weights.astype(jnp.float32).T
    zeros = jnp.zeros((WROW, WCOL), jnp.float32)

    w2 = _sc_scatter(idx_flat, w32, tau_t, zeros)

    nout = N_NEURONS * N_SYN
    out = pl.pallas_call(
        _mm_body,
        grid=((nout + _BN - 1) // _BN,),
        in_specs=[
            pl.BlockSpec((bt, WCOL), lambda j: (0, 0)),
            pl.BlockSpec((_BN, WCOL), lambda j: (j, 0)),
        ],
        out_specs=pl.BlockSpec((_BN, bt), lambda j: (j, 0)),
        out_shape=jax.ShapeDtypeStruct((nout, bt), jnp.float32),
    )(spikes, w2)

    return jnp.transpose(out).reshape(bsz, bt, nout)


# speedup vs baseline: 2.7946x; 1.7250x over previous
import jax
import jax.numpy as jnp
from jax import lax
from jax.experimental import pallas as pl
from jax.experimental.pallas import tpu as pltpu, tpu_sc as plsc

N_NEURONS = 10000
N_BKG = 100
NNZ = 160000
N_SYN = 5
BKG_RATE = 250

NC, NS = 2, 16
NW = NC * NS
K = 160
NPL = K // 16
N_PAD = 10240
NCHUNK = N_PAD // K
CHUNKS_PER_W = NCHUNK // NW
BLKN = 1664
NBIN = 1040
WROW = K * N_SYN
WCOL = 128
PPT = NNZ // NS
PPW = PPT // 2
ASTEP = (PPW + 15) // 16
LA = 5376


def _bin10(r):
    return lax.shift_right_logical(r * 52429, 19)


def _sc_body(idx_hbm, w_hbm, tau_hbm, zeros_hbm, out_hbm, shared):
    cid = lax.axis_index("c")
    sid = lax.axis_index("s")
    wid = sid * NC + cid
    lane = lax.iota(jnp.int32, 16)

    if True:
        def phase_a(idx2_v, rp_loc):
            fullv = jnp.full((16,), NNZ, jnp.int32)
            def init(v, _):
                rp_loc[pl.ds(v * 16, 16)] = fullv
                return 0
            lax.fori_loop(0, NBIN // 16, init, 0)

            z0 = jnp.zeros((16,), jnp.int32)
            for h in range(2):
                pbase = sid * PPT + h * PPW
                pend = pbase + PPW
                g0p = pl.multiple_of(
                    jnp.minimum(jnp.maximum(pbase - 128, 0) & ~jnp.int32(127),
                                NNZ - LA), 128)
                pltpu.sync_copy(idx_hbm.at[:, pl.ds(g0p, LA)], idx2_v)

                def scan(i, _, g0p=g0p, pbase=pbase, pend=pend):
                    p = pbase + i * 16 + lane
                    lpc = jnp.minimum(p - g0p, LA - 1)
                    r = plsc.load_gather(idx2_v, [z0, lpc])
                    rprev = plsc.load_gather(
                        idx2_v, [z0, jnp.maximum(lpc - 1, 0)])
                    q = _bin10(r)
                    qprev = jnp.where(p == 0, -1, _bin10(rprev))
                    m = (q != qprev) & (p < pend)
                    plsc.store_scatter(rp_loc, [q], p, mask=m)
                    return 0

                lax.fori_loop(0, ASTEP, scan, 0)

            pltpu.sync_copy(
                rp_loc,
                shared.at[pl.ds(pl.multiple_of(sid * NBIN, 8), NBIN)])

        pl.run_scoped(phase_a,
                      pltpu.VMEM((2, LA), jnp.int32),
                      pltpu.VMEM((NBIN,), jnp.int32))
        plsc.subcore_barrier()

        @pl.when(sid == 0)
        def _reduce():
            def phase_b(stage_v, fin_v):
                pltpu.sync_copy(shared, stage_v)

                def redv(v, _):
                    acc = stage_v[pl.ds(v * 16, 16)]
                    for t in range(1, NS):
                        acc = jnp.minimum(
                            acc, stage_v[pl.ds(t * NBIN + v * 16, 16)])
                    fin_v[pl.ds(v * 16, 16)] = acc
                    return 0
                lax.fori_loop(0, NBIN // 16, redv, 0)

                def sufv(i, carry):
                    v = NBIN // 16 - 1 - i
                    seg = fin_v[pl.ds(v * 16, 16)]
                    rs = lax.rev(seg, (0,))
                    run = -plsc.cummax(-rs)
                    comb = jnp.minimum(run, carry)
                    fin_v[pl.ds(v * 16, 16)] = lax.rev(comb, (0,))
                    return jnp.min(comb)
                lax.fori_loop(0, NBIN // 16, sufv, jnp.int32(NNZ))

                pltpu.sync_copy(fin_v, shared.at[pl.ds(0, NBIN)])

            pl.run_scoped(phase_b,
                          pltpu.VMEM((NS * NBIN,), jnp.int32),
                          pltpu.VMEM((NBIN,), jnp.int32))

        plsc.subcore_barrier()

        def phase_c(idx_v, w_v, tau_v, rp_v, wl_v):
            for it in range(CHUNKS_PER_W):
                chunk = wid + NW * it
                n0 = chunk * K

                pltpu.sync_copy(
                    shared.at[pl.ds(pl.multiple_of(chunk * 16, 8), 24)],
                    rp_v)
                a = plsc.load_gather(rp_v, [lane])
                b = plsc.load_gather(rp_v, [lane + 1])
                p1 = jnp.max(b)

                pltpu.sync_copy(zeros_hbm, wl_v)

                ws0 = jnp.min(a) & ~jnp.int32(127)

                def window(ws_carry, a=a, b=b, n0=n0):
                    ws = pl.multiple_of(ws_carry, 128)
                    we = ws + BLKN
                    wd = pl.multiple_of(jnp.minimum(ws, NNZ - BLKN), 128)
                    pltpu.sync_copy(idx_hbm.at[:, pl.ds(wd, BLKN)], idx_v)
                    pltpu.sync_copy(w_hbm.at[pl.ds(wd, BLKN)], w_v)
                    pltpu.sync_copy(tau_hbm.at[:, pl.ds(wd, BLKN)], tau_v)
                    c0 = jnp.maximum(a, ws)
                    bmin = jnp.minimum(b, we)
                    steps = jnp.max(jnp.maximum(bmin - c0, 0))

                    def step(i, _, c0=c0, bmin=bmin, wd=wd, n0=n0):
                        ci = c0 + i
                        m = ci < bmin
                        off = jnp.minimum(ci - wd, BLKN - 1)
                        r16 = plsc.load_gather(
                            idx_v, [jnp.zeros((16,), jnp.int32), off])
                        c16 = plsc.load_gather(
                            idx_v, [jnp.ones((16,), jnp.int32), off])
                        w16 = plsc.load_gather(w_v, [off])
                        row = (r16 - n0) * N_SYN
                        for s in range(N_SYN):
                            t16 = plsc.load_gather(
                                tau_v, [jnp.full((16,), s, jnp.int32), off])
                            plsc.addupdate_scatter(
                                wl_v, [row + s, c16], w16 * t16, mask=m)
                        return 0

                    lax.fori_loop(0, steps, step, 0)
                    return ws + BLKN

                lax.while_loop(lambda ws, p1=p1: ws < p1, window, ws0)

                pltpu.sync_copy(
                    wl_v,
                    out_hbm.at[pl.ds(pl.multiple_of(chunk * WROW, 8), WROW)])

        pl.run_scoped(phase_c,
                      pltpu.VMEM((2, BLKN), jnp.int32),
                      pltpu.VMEM((BLKN,), jnp.float32),
                      pltpu.VMEM((N_SYN, BLKN), jnp.float32),
                      pltpu.VMEM((24,), jnp.int32),
                      pltpu.VMEM((WROW, WCOL), jnp.float32))

_sc_scatter = pl.kernel(
    _sc_body,
    out_type=jax.ShapeDtypeStruct((NCHUNK * WROW, WCOL), jnp.float32),
    mesh=plsc.VectorSubcoreMesh(core_axis_name="c", subcore_axis_name="s",
                                num_cores=NC, num_subcores=NS),
    compiler_params=pltpu.CompilerParams(needs_layout_passes=False),
    scratch_types=[pltpu.VMEM_SHARED((NS * NBIN,), jnp.int32)],
)


def _mm_body(s_ref, w_ref, o_ref):
    o_ref[...] = lax.dot_general(
        w_ref[...], s_ref[...], (((1,), (1,)), ((), ())),
        preferred_element_type=jnp.float32)


_BN = 2048


@jax.jit
def kernel(inp, indices, weights, tau_syn_weights):
    bsz, t, _ = inp.shape
    bt = bsz * t

    spikes = (jax.random.uniform(jax.random.key(42), (bsz, t, N_BKG))
              < BKG_RATE * 0.001).astype(jnp.float32).reshape(bt, N_BKG)
    spikes = jnp.pad(spikes, ((0, 0), (0, WCOL - N_BKG)))

    idx_t = indices.astype(jnp.int32).T
    w32 = weights.astype(jnp.float32)
    tau_t = tau_syn_weights.astype(jnp.float32).T
    zeros = jnp.zeros((WROW, WCOL), jnp.float32)

    w2 = _sc_scatter(idx_t, w32, tau_t, zeros)

    nout = N_NEURONS * N_SYN
    out = pl.pallas_call(
        _mm_body,
        grid=((nout + _BN - 1) // _BN,),
        in_specs=[
            pl.BlockSpec((bt, WCOL), lambda j: (0, 0)),
            pl.BlockSpec((_BN, WCOL), lambda j: (j, 0)),
        ],
        out_specs=pl.BlockSpec((_BN, bt), lambda j: (j, 0)),
        out_shape=jax.ShapeDtypeStruct((nout, bt), jnp.float32),
    )(spikes, w2)

    return jnp.transpose(out).reshape(bsz, bt, nout)


# speedup vs baseline: 3.1015x; 1.1098x over previous
import jax
import jax.numpy as jnp
from jax import lax
from jax.experimental import pallas as pl
from jax.experimental.pallas import tpu as pltpu, tpu_sc as plsc

N_NEURONS = 10000
N_BKG = 100
NNZ = 160000
N_SYN = 5
BKG_RATE = 250

NC, NS = 2, 16
NW = NC * NS
K = 112
NPL = K // 16
N_PAD = 10752
NCHUNK = N_PAD // K
CHUNKS_PER_W = NCHUNK // NW
BLKN = 3456
NBIN = 1552
WROW = K * N_SYN
WCOL = 128
PPT = NNZ // NS
PPW = PPT // 2
ASTEP = (PPW + 15) // 16
LA = 5376


def _bin7(r):
    return lax.shift_right_logical(r * 18725, 17)


def _sc_body(idx_hbm, w_hbm, tau_hbm, zeros_hbm, out_hbm, shared,
             dsem, zsem):
    cid = lax.axis_index("c")
    sid = lax.axis_index("s")
    wid = sid * NC + cid
    lane = lax.iota(jnp.int32, 16)

    if True:
        def phase_a(idx2_v, rp_loc):
            fullv = jnp.full((16,), NNZ, jnp.int32)
            def init(v, _):
                rp_loc[pl.ds(v * 16, 16)] = fullv
                return 0
            lax.fori_loop(0, NBIN // 16, init, 0)

            z0 = jnp.zeros((16,), jnp.int32)
            for h in range(2):
                pbase = sid * PPT + h * PPW
                pend = pbase + PPW
                g0p = pl.multiple_of(
                    jnp.minimum(jnp.maximum(pbase - 128, 0) & ~jnp.int32(127),
                                NNZ - LA), 128)
                pltpu.sync_copy(idx_hbm.at[:, pl.ds(g0p, LA)], idx2_v)

                def scan(i, _, g0p=g0p, pbase=pbase, pend=pend):
                    p = pbase + i * 16 + lane
                    lpc = jnp.minimum(p - g0p, LA - 1)
                    r = plsc.load_gather(idx2_v, [z0, lpc])
                    rprev = plsc.load_gather(
                        idx2_v, [z0, jnp.maximum(lpc - 1, 0)])
                    q = _bin7(r)
                    qprev = jnp.where(p == 0, -1, _bin7(rprev))
                    m = (q != qprev) & (p < pend)
                    plsc.store_scatter(rp_loc, [q], p, mask=m)
                    return 0

                lax.fori_loop(0, ASTEP, scan, 0)

            pltpu.sync_copy(
                rp_loc,
                shared.at[pl.ds(pl.multiple_of(sid * NBIN, 8), NBIN)])

        pl.run_scoped(phase_a,
                      pltpu.VMEM((2, LA), jnp.int32),
                      pltpu.VMEM((NBIN,), jnp.int32))
        plsc.subcore_barrier()

        @pl.when(sid == 0)
        def _reduce():
            def phase_b(stage_v, fin_v):
                pltpu.sync_copy(shared, stage_v)

                def redv(v, _):
                    acc = stage_v[pl.ds(v * 16, 16)]
                    for t in range(1, NS):
                        acc = jnp.minimum(
                            acc, stage_v[pl.ds(t * NBIN + v * 16, 16)])
                    fin_v[pl.ds(v * 16, 16)] = acc
                    return 0
                lax.fori_loop(0, NBIN // 16, redv, 0)

                def sufv(i, carry):
                    v = NBIN // 16 - 1 - i
                    seg = fin_v[pl.ds(v * 16, 16)]
                    rs = lax.rev(seg, (0,))
                    run = -plsc.cummax(-rs)
                    comb = jnp.minimum(run, carry)
                    fin_v[pl.ds(v * 16, 16)] = lax.rev(comb, (0,))
                    return jnp.min(comb)
                lax.fori_loop(0, NBIN // 16, sufv, jnp.int32(NNZ))

                pltpu.sync_copy(fin_v, shared.at[pl.ds(0, NBIN)])

            pl.run_scoped(phase_b,
                          pltpu.VMEM((NS * NBIN,), jnp.int32),
                          pltpu.VMEM((NBIN,), jnp.int32))

        plsc.subcore_barrier()

        def phase_c(idx_v, w_v, tau_v, rp_v, wl_v):
            for it in range(CHUNKS_PER_W):
                chunk = wid + NW * it
                n0 = chunk * K

                zcp = pltpu.async_copy(zeros_hbm, wl_v, zsem)
                pltpu.sync_copy(
                    shared.at[pl.ds(pl.multiple_of(chunk * 16, 8), 24)],
                    rp_v)
                a = plsc.load_gather(rp_v, [lane])
                b = plsc.load_gather(rp_v, [lane + 1])
                p1 = jnp.max(b)
                zcp.wait()

                ws0 = jnp.min(a) & ~jnp.int32(127)

                def window(ws_carry, a=a, b=b, n0=n0):
                    ws = pl.multiple_of(ws_carry, 128)
                    we = ws + BLKN
                    wd = pl.multiple_of(jnp.minimum(ws, NNZ - BLKN), 128)
                    cp1 = pltpu.async_copy(
                        idx_hbm.at[:, pl.ds(wd, BLKN)], idx_v, dsem)
                    cp2 = pltpu.async_copy(
                        w_hbm.at[pl.ds(wd, BLKN)], w_v, dsem)
                    cp3 = pltpu.async_copy(
                        tau_hbm.at[:, pl.ds(wd, BLKN)], tau_v, dsem)
                    c0 = jnp.maximum(a, ws)
                    bmin = jnp.minimum(b, we)
                    steps = jnp.max(jnp.maximum(bmin - c0, 0))
                    cp1.wait()
                    cp2.wait()
                    cp3.wait()

                    def step(i, _, c0=c0, bmin=bmin, wd=wd, n0=n0):
                        ci = c0 + i
                        m = ci < bmin
                        off = jnp.minimum(ci - wd, BLKN - 1)
                        r16 = plsc.load_gather(
                            idx_v, [jnp.zeros((16,), jnp.int32), off])
                        c16 = plsc.load_gather(
                            idx_v, [jnp.ones((16,), jnp.int32), off])
                        w16 = plsc.load_gather(w_v, [off])
                        row = (r16 - n0) * N_SYN
                        for s in range(N_SYN):
                            t16 = plsc.load_gather(
                                tau_v, [jnp.full((16,), s, jnp.int32), off])
                            plsc.addupdate_scatter(
                                wl_v, [row + s, c16], w16 * t16, mask=m)
                        return 0

                    lax.fori_loop(0, steps, step, 0)
                    return ws + BLKN

                lax.while_loop(lambda ws, p1=p1: ws < p1, window, ws0)

                pltpu.sync_copy(
                    wl_v,
                    out_hbm.at[pl.ds(pl.multiple_of(chunk * WROW, 8), WROW)])

        pl.run_scoped(phase_c,
                      pltpu.VMEM((2, BLKN), jnp.int32),
                      pltpu.VMEM((BLKN,), jnp.float32),
                      pltpu.VMEM((N_SYN, BLKN), jnp.float32),
                      pltpu.VMEM((24,), jnp.int32),
                      pltpu.VMEM((WROW, WCOL), jnp.float32))

_sc_scatter = pl.kernel(
    _sc_body,
    out_type=jax.ShapeDtypeStruct((NCHUNK * WROW, WCOL), jnp.float32),
    mesh=plsc.VectorSubcoreMesh(core_axis_name="c", subcore_axis_name="s",
                                num_cores=NC, num_subcores=NS),
    compiler_params=pltpu.CompilerParams(needs_layout_passes=False),
    scratch_types=[pltpu.VMEM_SHARED((NS * NBIN,), jnp.int32),
                   pltpu.SemaphoreType.DMA, pltpu.SemaphoreType.DMA],
)


def _mm_body(s_ref, w_ref, o_ref):
    o_ref[...] = lax.dot_general(
        w_ref[...], s_ref[...], (((1,), (1,)), ((), ())),
        preferred_element_type=jnp.float32)


_BN = 2048


@jax.jit
def kernel(inp, indices, weights, tau_syn_weights):
    bsz, t, _ = inp.shape
    bt = bsz * t

    spikes = (jax.random.uniform(jax.random.key(42), (bsz, t, N_BKG))
              < BKG_RATE * 0.001).astype(jnp.float32).reshape(bt, N_BKG)
    spikes = jnp.pad(spikes, ((0, 0), (0, WCOL - N_BKG)))

    idx_t = indices.astype(jnp.int32).T
    w32 = weights.astype(jnp.float32)
    tau_t = tau_syn_weights.astype(jnp.float32).T
    zeros = jnp.zeros((WROW, WCOL), jnp.float32)

    w2 = _sc_scatter(idx_t, w32, tau_t, zeros)

    nout = N_NEURONS * N_SYN
    out = pl.pallas_call(
        _mm_body,
        grid=((nout + _BN - 1) // _BN,),
        in_specs=[
            pl.BlockSpec((bt, WCOL), lambda j: (0, 0)),
            pl.BlockSpec((_BN, WCOL), lambda j: (j, 0)),
        ],
        out_specs=pl.BlockSpec((_BN, bt), lambda j: (j, 0)),
        out_shape=jax.ShapeDtypeStruct((nout, bt), jnp.float32),
    )(spikes, w2)

    return jnp.transpose(out).reshape(bsz, bt, nout)


# speedup vs baseline: 3.8733x; 1.2488x over previous
import jax
import jax.numpy as jnp
from jax import lax
from jax.experimental import pallas as pl
from jax.experimental.pallas import tpu as pltpu, tpu_sc as plsc

N_NEURONS = 10000
N_BKG = 100
NNZ = 160000
N_SYN = 5
BKG_RATE = 250

NC, NS = 2, 16
NW = NC * NS
K = 112
NPL = K // 16
N_PAD = 10752
NCHUNK = N_PAD // K
CHUNKS_PER_W = NCHUNK // NW
BLKN = 3456
NBIN = 1552
WROW = K * N_SYN
WCOL = 128
PPT = NNZ // NS
PPW = PPT // 2
ASTEP = (PPW + 15) // 16
LA = 5376


def _bin7(r):
    return lax.shift_right_logical(r * 18725, 17)


def _sc_body(idx_hbm, w_hbm, tau_hbm, out_hbm, shared, dsem):
    cid = lax.axis_index("c")
    sid = lax.axis_index("s")
    wid = sid * NC + cid
    lane = lax.iota(jnp.int32, 16)

    if True:
        def phase_a(idx2_v, rp_loc):
            fullv = jnp.full((16,), NNZ, jnp.int32)
            def init(v, _):
                rp_loc[pl.ds(v * 16, 16)] = fullv
                return 0
            lax.fori_loop(0, NBIN // 16, init, 0)

            z0 = jnp.zeros((16,), jnp.int32)
            for h in range(2):
                pbase = sid * PPT + h * PPW
                pend = pbase + PPW
                g0p = pl.multiple_of(
                    jnp.minimum(jnp.maximum(pbase - 128, 0) & ~jnp.int32(127),
                                NNZ - LA), 128)
                pltpu.sync_copy(idx_hbm.at[:, pl.ds(g0p, LA)], idx2_v)

                def scan(i, _, g0p=g0p, pbase=pbase, pend=pend):
                    p = pbase + i * 16 + lane
                    lpc = jnp.minimum(p - g0p, LA - 1)
                    r = plsc.load_gather(idx2_v, [z0, lpc])
                    rprev = plsc.load_gather(
                        idx2_v, [z0, jnp.maximum(lpc - 1, 0)])
                    q = _bin7(r)
                    qprev = jnp.where(p == 0, -1, _bin7(rprev))
                    m = (q != qprev) & (p < pend)
                    plsc.store_scatter(rp_loc, [q], p, mask=m)
                    return 0

                lax.fori_loop(0, ASTEP, scan, 0)

            pltpu.sync_copy(
                rp_loc,
                shared.at[pl.ds(pl.multiple_of(sid * NBIN, 8), NBIN)])

        pl.run_scoped(phase_a,
                      pltpu.VMEM((2, LA), jnp.int32),
                      pltpu.VMEM((NBIN,), jnp.int32))
        plsc.subcore_barrier()

        @pl.when(sid == 0)
        def _reduce():
            def phase_b(stage_v, fin_v):
                pltpu.sync_copy(shared, stage_v)

                def redv(v, _):
                    acc = stage_v[pl.ds(v * 16, 16)]
                    for t in range(1, NS):
                        acc = jnp.minimum(
                            acc, stage_v[pl.ds(t * NBIN + v * 16, 16)])
                    fin_v[pl.ds(v * 16, 16)] = acc
                    return 0
                lax.fori_loop(0, NBIN // 16, redv, 0)

                def sufv(i, carry):
                    v = NBIN // 16 - 1 - i
                    seg = fin_v[pl.ds(v * 16, 16)]
                    rs = lax.rev(seg, (0,))
                    run = -plsc.cummax(-rs)
                    comb = jnp.minimum(run, carry)
                    fin_v[pl.ds(v * 16, 16)] = lax.rev(comb, (0,))
                    return jnp.min(comb)
                lax.fori_loop(0, NBIN // 16, sufv, jnp.int32(NNZ))

                pltpu.sync_copy(fin_v, shared.at[pl.ds(0, NBIN)])

            pl.run_scoped(phase_b,
                          pltpu.VMEM((NS * NBIN,), jnp.int32),
                          pltpu.VMEM((NBIN,), jnp.int32))

        plsc.subcore_barrier()

        def phase_c(idx_v, w_v, tau_v, rp_v, wl_v):
            for it in range(CHUNKS_PER_W):
                chunk = wid + NW * it
                n0 = chunk * K

                pltpu.sync_copy(
                    shared.at[pl.ds(pl.multiple_of(chunk * 16, 8), 24)],
                    rp_v)
                a = plsc.load_gather(rp_v, [lane])
                b = plsc.load_gather(rp_v, [lane + 1])
                p1 = jnp.max(b)

                ws0 = jnp.min(a) & ~jnp.int32(127)

                def window(ws_carry, a=a, b=b, n0=n0):
                    ws = pl.multiple_of(ws_carry, 128)
                    we = ws + BLKN
                    wd = pl.multiple_of(jnp.minimum(ws, NNZ - BLKN), 128)
                    cp1 = pltpu.async_copy(
                        idx_hbm.at[:, pl.ds(wd, BLKN)], idx_v, dsem)
                    cp2 = pltpu.async_copy(
                        w_hbm.at[pl.ds(wd, BLKN)], w_v, dsem)
                    cp3 = pltpu.async_copy(
                        tau_hbm.at[:, pl.ds(wd, BLKN)], tau_v, dsem)
                    c0 = jnp.maximum(a, ws)
                    bmin = jnp.minimum(b, we)
                    steps = jnp.max(jnp.maximum(bmin - c0, 0))
                    @pl.when(ws == ws0)
                    def _zero(ws=ws):
                        zf = jnp.zeros((16,), jnp.float32)
                        def zrow(rr, _):
                            for j in range(WCOL // 16):
                                wl_v[rr, pl.ds(j * 16, 16)] = zf
                            return 0
                        lax.fori_loop(0, WROW, zrow, 0)

                    cp1.wait()
                    cp2.wait()
                    cp3.wait()

                    def step(i, _, c0=c0, bmin=bmin, wd=wd, n0=n0):
                        ci = c0 + i
                        m = ci < bmin
                        off = jnp.minimum(ci - wd, BLKN - 1)
                        r16 = plsc.load_gather(
                            idx_v, [jnp.zeros((16,), jnp.int32), off])
                        c16 = plsc.load_gather(
                            idx_v, [jnp.ones((16,), jnp.int32), off])
                        w16 = plsc.load_gather(w_v, [off])
                        row = (r16 - n0) * N_SYN
                        for s in range(N_SYN):
                            t16 = plsc.load_gather(
                                tau_v, [jnp.full((16,), s, jnp.int32), off])
                            plsc.addupdate_scatter(
                                wl_v, [row + s, c16], w16 * t16, mask=m)
                        return 0

                    lax.fori_loop(0, steps, step, 0)
                    return ws + BLKN

                lax.while_loop(lambda ws, p1=p1: ws < p1, window, ws0)

                pltpu.sync_copy(
                    wl_v,
                    out_hbm.at[pl.ds(pl.multiple_of(chunk * WROW, 8), WROW)])

        pl.run_scoped(phase_c,
                      pltpu.VMEM((2, BLKN), jnp.int32),
                      pltpu.VMEM((BLKN,), jnp.float32),
                      pltpu.VMEM((N_SYN, BLKN), jnp.float32),
                      pltpu.VMEM((24,), jnp.int32),
                      pltpu.VMEM((WROW, WCOL), jnp.float32))

_sc_scatter = pl.kernel(
    _sc_body,
    out_type=jax.ShapeDtypeStruct((NCHUNK * WROW, WCOL), jnp.float32),
    mesh=plsc.VectorSubcoreMesh(core_axis_name="c", subcore_axis_name="s",
                                num_cores=NC, num_subcores=NS),
    compiler_params=pltpu.CompilerParams(needs_layout_passes=False),
    scratch_types=[pltpu.VMEM_SHARED((NS * NBIN,), jnp.int32),
                   pltpu.SemaphoreType.DMA],
)


def _mm_body(s_ref, w_ref, o_ref):
    o_ref[...] = lax.dot_general(
        w_ref[...], s_ref[...], (((1,), (1,)), ((), ())),
        preferred_element_type=jnp.float32)


_BN = 2048


@jax.jit
def kernel(inp, indices, weights, tau_syn_weights):
    bsz, t, _ = inp.shape
    bt = bsz * t

    spikes = (jax.random.uniform(jax.random.key(42), (bsz, t, N_BKG))
              < BKG_RATE * 0.001).astype(jnp.float32).reshape(bt, N_BKG)
    spikes = jnp.pad(spikes, ((0, 0), (0, WCOL - N_BKG)))

    idx_t = indices.astype(jnp.int32).T
    w32 = weights.astype(jnp.float32)
    tau_t = tau_syn_weights.astype(jnp.float32).T

    w2 = _sc_scatter(idx_t, w32, tau_t)

    nout = N_NEURONS * N_SYN
    out = pl.pallas_call(
        _mm_body,
        grid=((nout + _BN - 1) // _BN,),
        in_specs=[
            pl.BlockSpec((bt, WCOL), lambda j: (0, 0)),
            pl.BlockSpec((_BN, WCOL), lambda j: (j, 0)),
        ],
        out_specs=pl.BlockSpec((_BN, bt), lambda j: (j, 0)),
        out_shape=jax.ShapeDtypeStruct((nout, bt), jnp.float32),
    )(spikes, w2)

    return jnp.transpose(out).reshape(bsz, bt, nout)
